# baseline (speedup 1.0000x reference)
import jax
import jax.numpy as jnp
from jax import lax
from jax.experimental import pallas as pl
from jax.experimental.pallas import tpu as pltpu
from jax.experimental.pallas import tpu_sc as plsc
import numpy as np

CLS_ID = 101
SEP_ID = 102
MAX_LENGTH = 4096
B = 16
LQ = 64
HALF = MAX_LENGTH // 2
NCHUNK = 2 * B


def _body(q_hbm, d_hbm, qlens_hbm, dlens_hbm,
          tok_hbm, mask_hbm, pid_hbm,
          q_v, d_v, qlens_v, dlens_v, tok_v, mask_v, pid_v):
    nc = 2
    wid = lax.axis_index("s") * nc + lax.axis_index("c")
    row = wid // 2
    half = wid % 2
    base = (half * HALF).astype(jnp.int32)

    pltpu.sync_copy(qlens_hbm, qlens_v)
    pltpu.sync_copy(dlens_hbm, dlens_v)
    pltpu.sync_copy(q_hbm.at[row], q_v)
    pltpu.sync_copy(d_hbm.at[row], d_v)

    row_v = jnp.full((16,), row, jnp.int32)
    qlen = plsc.load_gather(qlens_v, [row_v])
    dlen = plsc.load_gather(dlens_v, [row_v])
    dlen_eff = jnp.minimum(dlen, np.int32(MAX_LENGTH - 2) - qlen)
    qoff = qlen + np.int32(2)
    total = qoff + dlen_eff
    lane = lax.iota(jnp.int32, 16)
    lane2 = lane * np.int32(2)
    zero_v = jnp.zeros((16,), jnp.int32)

    def step(i, _):
        p = lane + base + i * np.int32(16)
        q_idx = jnp.clip(p - np.int32(1), np.int32(0), np.int32(LQ - 1))
        q_tok = plsc.load_gather(q_v, [q_idx * np.int32(2)])
        d_idx = jnp.clip(p - qoff, np.int32(0), np.int32(MAX_LENGTH - 1))
        d_tok = plsc.load_gather(d_v, [d_idx * np.int32(2)])
        in_seq = p < total
        tok = jnp.where(p == np.int32(0), np.int32(CLS_ID),
              jnp.where(p <= qlen, q_tok,
              jnp.where(p == qoff - np.int32(1), np.int32(SEP_ID),
              jnp.where(in_seq, d_tok, np.int32(0)))))
        mask = jnp.where(in_seq, np.float32(1.0), np.float32(0.0))
        pid = jnp.where(p <= qlen, p,
              jnp.where(in_seq, p - qlen - np.int32(1), np.int32(0)))
        idx_e = i * np.int32(32) + lane2
        idx_o = idx_e + np.int32(1)
        plsc.store_scatter(tok_v, [idx_e], tok)
        plsc.store_scatter(tok_v, [idx_o], zero_v)
        plsc.store_scatter(pid_v, [idx_e], pid)
        plsc.store_scatter(pid_v, [idx_o], zero_v)
        mask_v[pl.ds(i * np.int32(16), 16)] = mask
        return 0

    lax.fori_loop(jnp.int32(0), jnp.int32(HALF // 16), step, 0)

    pltpu.sync_copy(tok_v, tok_hbm.at[wid])
    pltpu.sync_copy(mask_v, mask_hbm.at[wid])
    pltpu.sync_copy(pid_v, pid_hbm.at[wid])


_MESH = plsc.VectorSubcoreMesh(core_axis_name="c", subcore_axis_name="s")

_run = pl.kernel(
    _body,
    out_type=(
        jax.ShapeDtypeStruct((NCHUNK, 2 * HALF), jnp.int32),
        jax.ShapeDtypeStruct((NCHUNK, HALF), jnp.float32),
        jax.ShapeDtypeStruct((NCHUNK, 2 * HALF), jnp.int32),
    ),
    mesh=_MESH,
    compiler_params=pltpu.CompilerParams(needs_layout_passes=False),
    scratch_types=[
        pltpu.VMEM((2 * LQ,), jnp.int32),
        pltpu.VMEM((2 * MAX_LENGTH,), jnp.int32),
        pltpu.VMEM((B,), jnp.int32),
        pltpu.VMEM((B,), jnp.int32),
        pltpu.VMEM((2 * HALF,), jnp.int32),
        pltpu.VMEM((HALF,), jnp.float32),
        pltpu.VMEM((2 * HALF,), jnp.int32),
    ],
)


def kernel(query, document, q_lens, d_lens):
    q2 = lax.bitcast_convert_type(query, jnp.int32).reshape(B, 2 * LQ)
    d2 = lax.bitcast_convert_type(document, jnp.int32).reshape(B, 2 * MAX_LENGTH)
    tok2, mask, pid2 = _run(q2, d2, q_lens, d_lens)
    tok = lax.bitcast_convert_type(tok2.reshape(B, MAX_LENGTH, 2), jnp.int64)
    mask = mask.reshape(B, MAX_LENGTH)
    pid = lax.bitcast_convert_type(pid2.reshape(B, MAX_LENGTH, 2), jnp.int64)
    return tok, mask, pid


# speedup vs baseline: 6.5289x; 6.5289x over previous
import jax
import jax.numpy as jnp
from jax import lax
from jax.experimental import pallas as pl
from jax.experimental.pallas import tpu as pltpu
from jax.experimental.pallas import tpu_sc as plsc
import numpy as np

CLS_ID = 101
SEP_ID = 102
MAX_LENGTH = 4096
B = 16
LQ = 64
HALF = MAX_LENGTH // 2
NCHUNK = 2 * B
NPRE = 8


def _body(q_hbm, d_hbm, lens_hbm,
          tok_hbm, mask_hbm, pid_hbm,
          q_v, d_v, lens_v, tok_v, mask_v, pid_v, sem_in, sem_out):
    nc = 2
    wid = lax.axis_index("s") * nc + lax.axis_index("c")
    row = wid // 2
    half = wid % 2
    base = (half * HALF).astype(jnp.int32)

    c_lens = pltpu.async_copy(lens_hbm, lens_v, sem_in)
    c_q = pltpu.async_copy(q_hbm.at[row], q_v, sem_in)
    c_d = pltpu.async_copy(d_hbm.at[row], d_v, sem_in)
    c_lens.wait()

    row_v = jnp.full((16,), row, jnp.int32)
    qlen = plsc.load_gather(lens_v, [row_v])
    dlen = plsc.load_gather(lens_v, [row_v + np.int32(B)])
    dlen_eff = jnp.minimum(dlen, np.int32(MAX_LENGTH - 2) - qlen)
    qoff = qlen + np.int32(2)
    qlen1 = qlen + np.int32(1)
    total = qoff + dlen_eff
    lane = lax.iota(jnp.int32, 16)

    c_q.wait()
    c_d.wait()

    for i in range(NPRE):
        p = lane + base + np.int32(16 * i)
        q_idx = jnp.clip(p - np.int32(1), np.int32(0), np.int32(LQ - 1))
        q_tok = plsc.load_gather(q_v, [q_idx])
        d_idx = jnp.maximum(p - qoff, np.int32(0))
        d_tok = plsc.load_gather(d_v, [d_idx])
        in_seq = p < total
        tok = jnp.where(p == np.int32(0), np.int32(CLS_ID),
              jnp.where(p <= qlen, q_tok,
              jnp.where(p == qlen1, np.int32(SEP_ID),
              jnp.where(in_seq, d_tok, np.int32(0)))))
        mask = jnp.where(in_seq, np.float32(1.0), np.float32(0.0))
        pid = jnp.where(p <= qlen, p,
              jnp.where(in_seq, p - qlen1, np.int32(0)))
        tok_v[pl.ds(16 * i, 16)] = tok
        mask_v[pl.ds(16 * i, 16)] = mask
        pid_v[pl.ds(16 * i, 16)] = pid

    @plsc.parallel_loop(jnp.int32(NPRE), jnp.int32(HALF // 16),
                        step=jnp.int32(1), unroll=4)
    def _bulk(i):
        p = lane + base + i * np.int32(16)
        d_tok = plsc.load_gather(d_v, [p - qoff])
        in_seq = p < total
        off = i * np.int32(16)
        tok_v[pl.ds(off, 16)] = jnp.where(in_seq, d_tok, np.int32(0))
        mask_v[pl.ds(off, 16)] = jnp.where(in_seq, np.float32(1.0),
                                           np.float32(0.0))
        pid_v[pl.ds(off, 16)] = jnp.where(in_seq, p - qlen1, np.int32(0))

    o_tok = pltpu.async_copy(tok_v, tok_hbm.at[wid], sem_out)
    o_mask = pltpu.async_copy(mask_v, mask_hbm.at[wid], sem_out)
    o_pid = pltpu.async_copy(pid_v, pid_hbm.at[wid], sem_out)
    o_tok.wait()
    o_mask.wait()
    o_pid.wait()


_MESH = plsc.VectorSubcoreMesh(core_axis_name="c", subcore_axis_name="s")

_run = pl.kernel(
    _body,
    out_type=(
        jax.ShapeDtypeStruct((NCHUNK, HALF), jnp.int32),
        jax.ShapeDtypeStruct((NCHUNK, HALF), jnp.float32),
        jax.ShapeDtypeStruct((NCHUNK, HALF), jnp.int32),
    ),
    mesh=_MESH,
    compiler_params=pltpu.CompilerParams(needs_layout_passes=False),
    scratch_types=[
        pltpu.VMEM((LQ,), jnp.int32),
        pltpu.VMEM((MAX_LENGTH,), jnp.int32),
        pltpu.VMEM((2 * B,), jnp.int32),
        pltpu.VMEM((HALF,), jnp.int32),
        pltpu.VMEM((HALF,), jnp.float32),
        pltpu.VMEM((HALF,), jnp.int32),
        pltpu.SemaphoreType.DMA,
        pltpu.SemaphoreType.DMA,
    ],
)


def kernel(query, document, q_lens, d_lens):
    q32 = query.astype(jnp.int32)
    d32 = document.astype(jnp.int32)
    lens = jnp.concatenate([q_lens, d_lens])
    tok, mask, pid = _run(q32, d32, lens)
    tok = tok.reshape(B, MAX_LENGTH).astype(query.dtype)
    mask = mask.reshape(B, MAX_LENGTH)
    pid = pid.reshape(B, MAX_LENGTH).astype(jnp.int64)
    return tok, mask, pid
